# Initial kernel scaffold; baseline (speedup 1.0000x reference)
import jax
import jax.numpy as jnp
from jax.experimental import pallas as pl


def kernel(x, edge_index, W1, a1s, a1d, b1, gn1_w, gn1_b, gn1_ms, W2, a2s, a2d, b2, gn2_w, gn2_b, gn2_ms, W3, a3s, a3d, b3):
    raise NotImplementedError("write your pallas kernel here")


# speedup vs baseline: 1.0001x; 1.0001x over previous
import jax
import jax.numpy as jnp
from jax.experimental import pallas as pl

N_NODES = 100000
HEADS = 4


def _gat_conv(x, edge_index, W, att_src, att_dst, bias, heads, out_ch, concat):
    src = edge_index[0]
    dst = edge_index[1]
    N = x.shape[0]
    h = (x @ W).reshape(N, heads, out_ch)
    a_s = (h * att_src[None, :, :]).sum(-1)
    a_d = (h * att_dst[None, :, :]).sum(-1)
    e = a_s[src] + a_d[dst]
    e = jnp.where(e > 0, e, 0.2 * e)
    emax = jax.ops.segment_max(e, dst, num_segments=N)
    emax = jnp.where(jnp.isfinite(emax), emax, 0.0)
    ee = jnp.exp(e - emax[dst])
    denom = jax.ops.segment_sum(ee, dst, num_segments=N)
    alpha = ee / (denom[dst] + 1e-16)
    msg = h[src] * alpha[:, :, None]
    out = jax.ops.segment_sum(msg, dst, num_segments=N)
    if concat:
        out = out.reshape(N, heads * out_ch)
    else:
        out = out.mean(axis=1)
    return out + bias


def _graph_norm(x, weight, bias, mean_scale, eps=1e-5):
    mean = x.mean(axis=0, keepdims=True)
    xc = x - mean * mean_scale
    var = (xc * xc).mean(axis=0, keepdims=True)
    return weight * xc / jnp.sqrt(var + eps) + bias


def _elu(x, a=1.0):
    return jnp.where(x > 0, x, a * jnp.expm1(x))


def _bias_kernel(x_ref, b_ref, o_ref):
    o_ref[...] = x_ref[...] + b_ref[...]


def _bias_add(x, b):
    return pl.pallas_call(
        _bias_kernel,
        out_shape=jax.ShapeDtypeStruct(x.shape, x.dtype),
        grid=(100,),
        in_specs=[
            pl.BlockSpec((1000, x.shape[1]), lambda i: (i, 0)),
            pl.BlockSpec((1, x.shape[1]), lambda i: (0, 0)),
        ],
        out_specs=pl.BlockSpec((1000, x.shape[1]), lambda i: (i, 0)),
    )(x, b[None, :])


def kernel(x, edge_index, W1, a1s, a1d, b1, gn1_w, gn1_b, gn1_ms, W2, a2s, a2d, b2, gn2_w, gn2_b, gn2_ms, W3, a3s, a3d, b3):
    z = jnp.zeros((HEADS * 64,), jnp.float32)
    h = _gat_conv(x, edge_index, W1, a1s, a1d, z, HEADS, 64, True)
    h = _bias_add(h, b1)
    h = _graph_norm(h, gn1_w, gn1_b, gn1_ms)
    h = _elu(h)
    h = _gat_conv(h, edge_index, W2, a2s, a2d, b2, HEADS, 96, True)
    h = _graph_norm(h, gn2_w, gn2_b, gn2_ms)
    h = _elu(h)
    out = _gat_conv(h, edge_index, W3, a3s, a3d, b3, 1, 48, False)
    return out


# speedup vs baseline: 16.4074x; 16.4059x over previous
import functools
import jax
import jax.numpy as jnp
from jax import lax
from jax.experimental import pallas as pl
from jax.experimental.pallas import tpu as pltpu
from jax.experimental.pallas import tpu_sc as plsc

N = 100000
E = 1600000
NC, NS, L = 2, 16, 16
NW = NC * NS
SPAN = 256
SHIFT = 8
NB = (N + SPAN - 1) // SPAN
NBP = 512
NP = NB * SPAN
EPW = E // NW
EPAD = 128
HCH = 8192
CH = 512
RB = 1000
GRID = N // RB

_mesh = plsc.VectorSubcoreMesh(core_axis_name="c", subcore_axis_name="s")
_sc_params = pltpu.CompilerParams(use_tc_tiling_on_sc=False)

_CHUNKS = []
_o = 0
while _o < EPW:
    _CHUNKS.append((_o, min(HCH, EPW - _o)))
    _o += HCH


def _wid():
    return lax.axis_index("s") * NC + lax.axis_index("c")


def _fill_iota(idx_v, base, count):
    def body(j, _):
        idx_v[pl.ds(j * L, L)] = (jnp.full((L,), base + j * L, jnp.int32)
                                  + lax.iota(jnp.int32, L))
        return 0
    lax.fori_loop(0, count // L, body, 0)


@functools.partial(
    pl.kernel,
    out_type=jax.ShapeDtypeStruct((NW, NBP), jnp.int32),
    mesh=_mesh,
    compiler_params=_sc_params,
    scratch_types=[
        pltpu.VMEM((HCH,), jnp.int32),
        pltpu.VMEM((HCH,), jnp.int32),
        pltpu.VMEM((NBP,), jnp.int32),
        pltpu.SemaphoreType.DMA,
    ],
)
def _hist_k(dst, cnt_out, idx_v, dst_v, cnt_v, sem):
    wid = _wid()
    ebase = wid * EPW

    def zc(j, _):
        cnt_v[pl.ds(j * L, L)] = jnp.zeros((L,), jnp.int32)
        return 0
    lax.fori_loop(0, NBP // L, zc, 0)

    for (coff, clen) in _CHUNKS:
        _fill_iota(idx_v, ebase + coff, clen)
        pltpu.async_copy(dst.at[idx_v.at[pl.ds(0, clen)]],
                         dst_v.at[pl.ds(0, clen)], sem).wait()

        def hist(i, _):
            d = dst_v[pl.ds(i, 1)][0]
            b = d >> SHIFT
            cnt_v[pl.ds(b, 1)] = cnt_v[pl.ds(b, 1)] + 1
            return 0
        lax.fori_loop(0, clen, hist, 0)

    pltpu.sync_copy(cnt_v, cnt_out.at[wid])


def _off_body(cnt_ref, off_ref, bptr_ref):
    ci = cnt_ref[...]
    tot = jnp.sum(ci, axis=0, keepdims=True)
    s = tot
    k = 1
    while k < NBP:
        s = s + jnp.concatenate(
            [jnp.zeros((1, k), jnp.int32), s[:, :-k]], axis=1)
        k *= 2
    bptr = s - tot
    wp = ci
    k = 1
    while k < NW:
        wp = wp + jnp.concatenate(
            [jnp.zeros((k, NBP), jnp.int32), wp[:-k, :]], axis=0)
        k *= 2
    off_ref[...] = bptr + (wp - ci)
    bptr_ref[...] = bptr


def _offsets(counts):
    return pl.pallas_call(
        _off_body,
        out_shape=[jax.ShapeDtypeStruct((NW, NBP), jnp.int32),
                   jax.ShapeDtypeStruct((1, NBP), jnp.int32)],
    )(counts)


@functools.partial(
    pl.kernel,
    out_type=[jax.ShapeDtypeStruct((E + EPAD,), jnp.int32),
              jax.ShapeDtypeStruct((E + EPAD,), jnp.int32)],
    mesh=_mesh,
    compiler_params=_sc_params,
    scratch_types=[
        pltpu.VMEM((HCH,), jnp.int32),
        pltpu.VMEM((HCH,), jnp.int32),
        pltpu.VMEM((HCH,), jnp.int32),
        pltpu.VMEM((CH,), jnp.int32),
        pltpu.VMEM((NBP,), jnp.int32),
        pltpu.VMEM((NBP,), jnp.int32),
        pltpu.SemaphoreType.DMA,
        pltpu.SemaphoreType.DMA,
    ],
)
def _place_k(src, dst, off, out_src, out_dst, idx_v, src_v, dst_v, pos_v,
             cnt_v, off_v, sem1, sem2):
    wid = _wid()
    ebase = wid * EPW

    def zc(j, _):
        cnt_v[pl.ds(j * L, L)] = jnp.zeros((L,), jnp.int32)
        return 0
    lax.fori_loop(0, NBP // L, zc, 0)
    pltpu.sync_copy(off.at[wid], off_v)

    @pl.when(wid == 0)
    def _():
        _fill_iota(idx_v, E, EPAD)
        pltpu.async_copy(cnt_v.at[pl.ds(0, EPAD)],
                         out_src.at[idx_v.at[pl.ds(0, EPAD)]], sem1).wait()
        pltpu.async_copy(cnt_v.at[pl.ds(0, EPAD)],
                         out_dst.at[idx_v.at[pl.ds(0, EPAD)]], sem2).wait()

    for (coff, clen) in _CHUNKS:
        _fill_iota(idx_v, ebase + coff, clen)
        cpa = pltpu.async_copy(src.at[idx_v.at[pl.ds(0, clen)]],
                               src_v.at[pl.ds(0, clen)], sem1)
        cpb = pltpu.async_copy(dst.at[idx_v.at[pl.ds(0, clen)]],
                               dst_v.at[pl.ds(0, clen)], sem2)
        cpa.wait()
        cpb.wait()

        def place_chunk(base, size):
            def place(i, _):
                d = dst_v[pl.ds(base + i, 1)][0]
                b = d >> SHIFT
                cur = cnt_v[pl.ds(b, 1)][0]
                pos_v[pl.ds(i, 1)] = off_v[pl.ds(b, 1)] + cur
                cnt_v[pl.ds(b, 1)] = cnt_v[pl.ds(b, 1)] + 1
                return 0
            lax.fori_loop(0, size, place, 0)
            cp1 = pltpu.async_copy(src_v.at[pl.ds(base, size)],
                                   out_src.at[pos_v.at[pl.ds(0, size)]], sem1)
            cp2 = pltpu.async_copy(dst_v.at[pl.ds(base, size)],
                                   out_dst.at[pos_v.at[pl.ds(0, size)]], sem2)
            cp1.wait()
            cp2.wait()

        n_full = clen // CH
        tail = clen % CH

        def sub(k, _):
            place_chunk(k * CH, CH)
            return 0
        lax.fori_loop(0, n_full, sub, 0)
        if tail:
            place_chunk(n_full * CH, tail)


def _make_edge_kernel(D, H, F, C):
    nvr = D // L

    @functools.partial(
        pl.kernel,
        out_type=jax.ShapeDtypeStruct((NP, D), jnp.float32),
        mesh=_mesh,
        compiler_params=_sc_params,
        scratch_types=[
            pltpu.VMEM((NBP,), jnp.int32),
            pltpu.VMEM((L,), jnp.float32),
            pltpu.VMEM((C,), jnp.int32),
            pltpu.VMEM((C,), jnp.int32),
            pltpu.VMEM((C,), jnp.int32),
            pltpu.VMEM((C, D), jnp.float32),
            pltpu.VMEM((C, L), jnp.float32),
            pltpu.VMEM((C, L), jnp.float32),
            pltpu.VMEM((SPAN, D), jnp.float32),
            pltpu.VMEM((SPAN, L), jnp.float32),
            pltpu.SemaphoreType.DMA,
            pltpu.SemaphoreType.DMA,
            pltpu.SemaphoreType.DMA,
        ],
    )
    def k(hfeat, asrc, adst, ssrc, sdst, bptr, cvec, out,
          bptr_v, cv, eidx, ssrc_v, sdst_v, rows, asv, adv, acc, den,
          s1, s2, s3):
        wid = _wid()
        pltpu.sync_copy(bptr.at[0], bptr_v)
        pltpu.sync_copy(cvec.at[0], cv)
        cvv = cv[...]
        nown = (NB - wid + NW - 1) // NW

        def bucket_body(kk, _):
            b = wid + kk * NW
            nbase = b * SPAN

            def zacc(i, _):
                r = i // nvr
                j = i % nvr
                acc[r, pl.ds(j * L, L)] = jnp.zeros((L,), jnp.float32)
                return 0
            lax.fori_loop(0, SPAN * nvr, zacc, 0)

            def zden(r, _):
                den[r, :] = jnp.zeros((L,), jnp.float32)
                return 0
            lax.fori_loop(0, SPAN, zden, 0)

            e0 = bptr_v[pl.ds(b, 1)][0]
            e1 = bptr_v[pl.ds(b + 1, 1)][0]
            ec = e1 - e0
            nch = (ec + C - 1) // C

            def chunk_body(ch, _):
                ebase2 = e0 + ch * C
                _fill_iota(eidx, ebase2, C)
                cpa = pltpu.async_copy(ssrc.at[eidx], ssrc_v, s1)
                cpb = pltpu.async_copy(sdst.at[eidx], sdst_v, s2)
                cpa.wait()
                cpb.wait()
                cp1 = pltpu.async_copy(hfeat.at[ssrc_v], rows, s1)
                cp2 = pltpu.async_copy(asrc.at[ssrc_v], asv, s2)
                cp3 = pltpu.async_copy(adst.at[sdst_v], adv, s3)
                cp1.wait()
                cp2.wait()
                cp3.wait()
                rem = jnp.minimum(C, ec - ch * C)

                def edge_body(e, _):
                    r = sdst_v[pl.ds(e, 1)][0] - nbase
                    ev = asv[e, :] + adv[e, :]
                    ev = jnp.maximum(ev, 0.2 * ev)
                    wv = jnp.exp(ev - cvv)
                    den[r, :] = den[r, :] + wv
                    for h in range(H):
                        bc = jnp.full((L,), wv[h])
                        for j in range(F // L):
                            o = h * F + j * L
                            acc[r, pl.ds(o, L)] = (acc[r, pl.ds(o, L)]
                                                   + bc * rows[e, pl.ds(o, L)])
                    return 0
                lax.fori_loop(0, rem, edge_body, 0)
                return 0
            lax.fori_loop(0, nch, chunk_body, 0)

            def flush_body(r, _):
                drow = den[r, :]
                for h in range(H):
                    dv = jnp.full((L,), drow[h]) + 1e-16
                    inv = 1.0 / dv
                    for j in range(F // L):
                        o = h * F + j * L
                        acc[r, pl.ds(o, L)] = acc[r, pl.ds(o, L)] * inv
                return 0
            lax.fori_loop(0, SPAN, flush_body, 0)
            pltpu.sync_copy(acc, out.at[pl.ds(nbase, SPAN)])
            return 0
        lax.fori_loop(0, nown, bucket_body, 0)

    return k


_edge_k1 = _make_edge_kernel(256, 4, 64, 64)
_edge_k2 = _make_edge_kernel(384, 4, 96, 48)
_edge_k3 = _make_edge_kernel(48, 1, 48, 128)


def _prep_body(x_ref, w_ref, asm_ref, adm_ref, h_ref, as_ref, ad_ref, c_ref,
               ms_ref, md_ref):
    i = pl.program_id(0)
    h = jnp.dot(x_ref[...], w_ref[...], preferred_element_type=jnp.float32)
    h_ref[...] = h
    a_s = jnp.dot(h, asm_ref[...], preferred_element_type=jnp.float32)
    a_d = jnp.dot(h, adm_ref[...], preferred_element_type=jnp.float32)
    as_ref[...] = a_s
    ad_ref[...] = a_d
    bs = jnp.max(a_s, axis=0, keepdims=True)
    bd = jnp.max(a_d, axis=0, keepdims=True)

    @pl.when(i == 0)
    def _():
        ms_ref[...] = bs
        md_ref[...] = bd

    @pl.when(i > 0)
    def _():
        ms_ref[...] = jnp.maximum(ms_ref[...], bs)
        md_ref[...] = jnp.maximum(md_ref[...], bd)

    m = ms_ref[...] + md_ref[...]
    c_ref[...] = jnp.maximum(m, 0.2 * m)


def _prep(x, w, asm, adm, K, DOUT):
    return pl.pallas_call(
        _prep_body,
        grid=(GRID,),
        in_specs=[
            pl.BlockSpec((RB, K), lambda i: (i, 0)),
            pl.BlockSpec((K, DOUT), lambda i: (0, 0)),
            pl.BlockSpec((DOUT, 16), lambda i: (0, 0)),
            pl.BlockSpec((DOUT, 16), lambda i: (0, 0)),
        ],
        out_specs=[
            pl.BlockSpec((RB, DOUT), lambda i: (i, 0)),
            pl.BlockSpec((RB, 16), lambda i: (i, 0)),
            pl.BlockSpec((RB, 16), lambda i: (i, 0)),
            pl.BlockSpec((1, 16), lambda i: (0, 0)),
        ],
        out_shape=[
            jax.ShapeDtypeStruct((N, DOUT), jnp.float32),
            jax.ShapeDtypeStruct((N, 16), jnp.float32),
            jax.ShapeDtypeStruct((N, 16), jnp.float32),
            jax.ShapeDtypeStruct((1, 16), jnp.float32),
        ],
        scratch_shapes=[pltpu.VMEM((1, 16), jnp.float32),
                        pltpu.VMEM((1, 16), jnp.float32)],
    )(x, w, asm, adm)


def _stats_body(z_ref, b_ref, sum_ref, sq_ref):
    i = pl.program_id(0)
    t = z_ref[...] + b_ref[...]
    s1 = jnp.sum(t, axis=0, keepdims=True)
    s2 = jnp.sum(t * t, axis=0, keepdims=True)

    @pl.when(i == 0)
    def _():
        sum_ref[...] = s1
        sq_ref[...] = s2

    @pl.when(i > 0)
    def _():
        sum_ref[...] = sum_ref[...] + s1
        sq_ref[...] = sq_ref[...] + s2


def _stats(z, b, D):
    return pl.pallas_call(
        _stats_body,
        grid=(GRID,),
        in_specs=[
            pl.BlockSpec((RB, D), lambda i: (i, 0)),
            pl.BlockSpec((1, D), lambda i: (0, 0)),
        ],
        out_specs=[
            pl.BlockSpec((1, D), lambda i: (0, 0)),
            pl.BlockSpec((1, D), lambda i: (0, 0)),
        ],
        out_shape=[jax.ShapeDtypeStruct((1, D), jnp.float32),
                   jax.ShapeDtypeStruct((1, D), jnp.float32)],
    )(z, b)


def _apply_prep_body(z_ref, b_ref, gw_ref, gb_ref, gms_ref, s1_ref, s2_ref,
                     w_ref, asm_ref, adm_ref, h_ref, as_ref, ad_ref, c_ref,
                     ms_ref, md_ref):
    i = pl.program_id(0)
    t = z_ref[...] + b_ref[...]
    m = s1_ref[...] * (1.0 / N)
    mm = m * gms_ref[...]
    var = s2_ref[...] * (1.0 / N) - 2.0 * mm * m + mm * mm
    y = gw_ref[...] * (t - mm) / jnp.sqrt(var + 1e-5) + gb_ref[...]
    a = jnp.where(y > 0, y, jnp.exp(y) - 1.0)
    h = jnp.dot(a, w_ref[...], preferred_element_type=jnp.float32)
    h_ref[...] = h
    a_s = jnp.dot(h, asm_ref[...], preferred_element_type=jnp.float32)
    a_d = jnp.dot(h, adm_ref[...], preferred_element_type=jnp.float32)
    as_ref[...] = a_s
    ad_ref[...] = a_d
    bs = jnp.max(a_s, axis=0, keepdims=True)
    bd = jnp.max(a_d, axis=0, keepdims=True)

    @pl.when(i == 0)
    def _():
        ms_ref[...] = bs
        md_ref[...] = bd

    @pl.when(i > 0)
    def _():
        ms_ref[...] = jnp.maximum(ms_ref[...], bs)
        md_ref[...] = jnp.maximum(md_ref[...], bd)

    mx = ms_ref[...] + md_ref[...]
    c_ref[...] = jnp.maximum(mx, 0.2 * mx)


def _apply_prep(z, b, gw, gb, gms, s1, s2, w, asm, adm, D, DOUT):
    return pl.pallas_call(
        _apply_prep_body,
        grid=(GRID,),
        in_specs=[
            pl.BlockSpec((RB, D), lambda i: (i, 0)),
            pl.BlockSpec((1, D), lambda i: (0, 0)),
            pl.BlockSpec((1, D), lambda i: (0, 0)),
            pl.BlockSpec((1, D), lambda i: (0, 0)),
            pl.BlockSpec((1, D), lambda i: (0, 0)),
            pl.BlockSpec((1, D), lambda i: (0, 0)),
            pl.BlockSpec((1, D), lambda i: (0, 0)),
            pl.BlockSpec((D, DOUT), lambda i: (0, 0)),
            pl.BlockSpec((DOUT, 16), lambda i: (0, 0)),
            pl.BlockSpec((DOUT, 16), lambda i: (0, 0)),
        ],
        out_specs=[
            pl.BlockSpec((RB, DOUT), lambda i: (i, 0)),
            pl.BlockSpec((RB, 16), lambda i: (i, 0)),
            pl.BlockSpec((RB, 16), lambda i: (i, 0)),
            pl.BlockSpec((1, 16), lambda i: (0, 0)),
        ],
        out_shape=[
            jax.ShapeDtypeStruct((N, DOUT), jnp.float32),
            jax.ShapeDtypeStruct((N, 16), jnp.float32),
            jax.ShapeDtypeStruct((N, 16), jnp.float32),
            jax.ShapeDtypeStruct((1, 16), jnp.float32),
        ],
        scratch_shapes=[pltpu.VMEM((1, 16), jnp.float32),
                        pltpu.VMEM((1, 16), jnp.float32)],
    )(z, b, gw, gb, gms, s1, s2, w, asm, adm)


def _final_body(z_ref, b_ref, o_ref):
    o_ref[...] = z_ref[...] + b_ref[...]


def _final(z, b, D):
    return pl.pallas_call(
        _final_body,
        grid=(GRID,),
        in_specs=[
            pl.BlockSpec((RB, D), lambda i: (i, 0)),
            pl.BlockSpec((1, D), lambda i: (0, 0)),
        ],
        out_specs=pl.BlockSpec((RB, D), lambda i: (i, 0)),
        out_shape=jax.ShapeDtypeStruct((N, D), jnp.float32),
    )(z, b)


def _amap(a, H, F):
    m = jnp.zeros((H * F, 16), jnp.float32)
    for h in range(H):
        m = m.at[h * F:(h + 1) * F, h].set(a[h])
    return m


def kernel(x, edge_index, W1, a1s, a1d, b1, gn1_w, gn1_b, gn1_ms,
           W2, a2s, a2d, b2, gn2_w, gn2_b, gn2_ms, W3, a3s, a3d, b3):
    src = edge_index[0]
    dst = edge_index[1]

    counts = _hist_k(dst)
    off, bptr = _offsets(counts)
    ssrc, sdst = _place_k(src, dst, off)

    xp = jnp.pad(x, ((0, 0), (0, 128 - 11)))
    w1p = jnp.pad(W1, ((0, 128 - 11), (0, 0)))
    h1, as1, ad1, c1 = _prep(xp, w1p, _amap(a1s, 4, 64), _amap(a1d, 4, 64),
                             128, 256)
    z1 = _edge_k1(h1, as1, ad1, ssrc, sdst, bptr, c1)

    s1a, s1b = _stats(z1, b1[None, :], 256)
    h2, as2, ad2, c2 = _apply_prep(z1, b1[None, :], gn1_w[None, :],
                                   gn1_b[None, :], gn1_ms[None, :], s1a, s1b,
                                   W2, _amap(a2s, 4, 96), _amap(a2d, 4, 96),
                                   256, 384)
    z2 = _edge_k2(h2, as2, ad2, ssrc, sdst, bptr, c2)

    s2a, s2b = _stats(z2, b2[None, :], 384)
    h3, as3, ad3, c3 = _apply_prep(z2, b2[None, :], gn2_w[None, :],
                                   gn2_b[None, :], gn2_ms[None, :], s2a, s2b,
                                   W3, _amap(a3s, 1, 48), _amap(a3d, 1, 48),
                                   384, 48)
    z3 = _edge_k3(h3, as3, ad3, ssrc, sdst, bptr, c3)

    return _final(z3, b3[None, :], 48)


# speedup vs baseline: 18.4677x; 1.1256x over previous
import functools
import jax
import jax.numpy as jnp
from jax import lax
from jax.experimental import pallas as pl
from jax.experimental.pallas import tpu as pltpu
from jax.experimental.pallas import tpu_sc as plsc

N = 100000
E = 1600000
NC, NS, L = 2, 16, 16
NW = NC * NS
SPAN = 256
SHIFT = 8
NB = (N + SPAN - 1) // SPAN
NBP = 512
NP = NB * SPAN
EPW = E // NW
EPAD = 128
HCH = 8192
CH = 512
RB = 1000
GRID = N // RB

_mesh = plsc.VectorSubcoreMesh(core_axis_name="c", subcore_axis_name="s")
_sc_params = pltpu.CompilerParams(use_tc_tiling_on_sc=False)

_CHUNKS = []
_o = 0
while _o < EPW:
    _CHUNKS.append((_o, min(HCH, EPW - _o)))
    _o += HCH


def _wid():
    return lax.axis_index("s") * NC + lax.axis_index("c")


def _fill_iota(idx_v, base, count):
    def body(j, _):
        idx_v[pl.ds(j * L, L)] = (jnp.full((L,), base + j * L, jnp.int32)
                                  + lax.iota(jnp.int32, L))
        return 0
    lax.fori_loop(0, count // L, body, 0)


@functools.partial(
    pl.kernel,
    out_type=jax.ShapeDtypeStruct((NW, NBP), jnp.int32),
    mesh=_mesh,
    compiler_params=_sc_params,
    scratch_types=[
        pltpu.VMEM((HCH,), jnp.int32),
        pltpu.VMEM((HCH,), jnp.int32),
        pltpu.VMEM((NBP,), jnp.int32),
        pltpu.SemaphoreType.DMA,
    ],
)
def _hist_k(dst, cnt_out, idx_v, dst_v, cnt_v, sem):
    wid = _wid()
    ebase = wid * EPW

    def zc(j, _):
        cnt_v[pl.ds(j * L, L)] = jnp.zeros((L,), jnp.int32)
        return 0
    lax.fori_loop(0, NBP // L, zc, 0)

    for (coff, clen) in _CHUNKS:
        _fill_iota(idx_v, ebase + coff, clen)
        pltpu.async_copy(dst.at[idx_v.at[pl.ds(0, clen)]],
                         dst_v.at[pl.ds(0, clen)], sem).wait()

        def hist(i, _):
            d = dst_v[pl.ds(i, 1)][0]
            b = d >> SHIFT
            cnt_v[pl.ds(b, 1)] = cnt_v[pl.ds(b, 1)] + 1
            return 0
        lax.fori_loop(0, clen, hist, 0)

    pltpu.sync_copy(cnt_v, cnt_out.at[wid])


def _off_body(cnt_ref, off_ref, bptr_ref):
    ci = cnt_ref[...]
    tot = jnp.sum(ci, axis=0, keepdims=True)
    s = tot
    k = 1
    while k < NBP:
        s = s + jnp.concatenate(
            [jnp.zeros((1, k), jnp.int32), s[:, :-k]], axis=1)
        k *= 2
    bptr = s - tot
    wp = ci
    k = 1
    while k < NW:
        wp = wp + jnp.concatenate(
            [jnp.zeros((k, NBP), jnp.int32), wp[:-k, :]], axis=0)
        k *= 2
    off_ref[...] = bptr + (wp - ci)
    bptr_ref[...] = bptr


def _offsets(counts):
    return pl.pallas_call(
        _off_body,
        out_shape=[jax.ShapeDtypeStruct((NW, NBP), jnp.int32),
                   jax.ShapeDtypeStruct((1, NBP), jnp.int32)],
    )(counts)


@functools.partial(
    pl.kernel,
    out_type=[jax.ShapeDtypeStruct((E + EPAD,), jnp.int32),
              jax.ShapeDtypeStruct((E + EPAD,), jnp.int32)],
    mesh=_mesh,
    compiler_params=_sc_params,
    scratch_types=[
        pltpu.VMEM((HCH,), jnp.int32),
        pltpu.VMEM((HCH,), jnp.int32),
        pltpu.VMEM((HCH,), jnp.int32),
        pltpu.VMEM((CH,), jnp.int32),
        pltpu.VMEM((NBP,), jnp.int32),
        pltpu.VMEM((NBP,), jnp.int32),
        pltpu.SemaphoreType.DMA,
        pltpu.SemaphoreType.DMA,
    ],
)
def _place_k(src, dst, off, out_src, out_dst, idx_v, src_v, dst_v, pos_v,
             cnt_v, off_v, sem1, sem2):
    wid = _wid()
    ebase = wid * EPW

    def zc(j, _):
        cnt_v[pl.ds(j * L, L)] = jnp.zeros((L,), jnp.int32)
        return 0
    lax.fori_loop(0, NBP // L, zc, 0)
    pltpu.sync_copy(off.at[wid], off_v)

    @pl.when(wid == 0)
    def _():
        _fill_iota(idx_v, E, EPAD)
        pltpu.async_copy(cnt_v.at[pl.ds(0, EPAD)],
                         out_src.at[idx_v.at[pl.ds(0, EPAD)]], sem1).wait()
        pltpu.async_copy(cnt_v.at[pl.ds(0, EPAD)],
                         out_dst.at[idx_v.at[pl.ds(0, EPAD)]], sem2).wait()

    for (coff, clen) in _CHUNKS:
        _fill_iota(idx_v, ebase + coff, clen)
        cpa = pltpu.async_copy(src.at[idx_v.at[pl.ds(0, clen)]],
                               src_v.at[pl.ds(0, clen)], sem1)
        cpb = pltpu.async_copy(dst.at[idx_v.at[pl.ds(0, clen)]],
                               dst_v.at[pl.ds(0, clen)], sem2)
        cpa.wait()
        cpb.wait()

        def place_chunk(base, size):
            def place(i, _):
                d = dst_v[pl.ds(base + i, 1)][0]
                b = d >> SHIFT
                cur = cnt_v[pl.ds(b, 1)][0]
                pos_v[pl.ds(i, 1)] = off_v[pl.ds(b, 1)] + cur
                cnt_v[pl.ds(b, 1)] = cnt_v[pl.ds(b, 1)] + 1
                return 0
            lax.fori_loop(0, size, place, 0)
            cp1 = pltpu.async_copy(src_v.at[pl.ds(base, size)],
                                   out_src.at[pos_v.at[pl.ds(0, size)]], sem1)
            cp2 = pltpu.async_copy(dst_v.at[pl.ds(base, size)],
                                   out_dst.at[pos_v.at[pl.ds(0, size)]], sem2)
            cp1.wait()
            cp2.wait()

        n_full = clen // CH
        tail = clen % CH

        def sub(k, _):
            place_chunk(k * CH, CH)
            return 0
        lax.fori_loop(0, n_full, sub, 0)
        if tail:
            place_chunk(n_full * CH, tail)


def _make_edge_kernel(D, H, F, C):
    nvr = D // L

    @functools.partial(
        pl.kernel,
        out_type=jax.ShapeDtypeStruct((NP, D), jnp.float32),
        mesh=_mesh,
        compiler_params=_sc_params,
        scratch_types=[
            pltpu.VMEM((NBP,), jnp.int32),
            pltpu.VMEM((L,), jnp.float32),
            pltpu.VMEM((C,), jnp.int32),
            pltpu.VMEM((C,), jnp.int32),
            pltpu.VMEM((C,), jnp.int32),
            pltpu.VMEM((C, D), jnp.float32),
            pltpu.VMEM((C, L), jnp.float32),
            pltpu.VMEM((C, L), jnp.float32),
            pltpu.VMEM((SPAN, D), jnp.float32),
            pltpu.VMEM((SPAN, L), jnp.float32),
            pltpu.SemaphoreType.DMA,
            pltpu.SemaphoreType.DMA,
            pltpu.SemaphoreType.DMA,
        ],
    )
    def k(hfeat, asrc, adst, ssrc, sdst, bptr, cvec, out,
          bptr_v, cv, eidx, ssrc_v, sdst_v, rows, asv, adv, acc, den,
          s1, s2, s3):
        wid = _wid()
        pltpu.sync_copy(bptr.at[0], bptr_v)
        pltpu.sync_copy(cvec.at[0], cv)
        cvv = cv[...]
        nown = (NB - wid + NW - 1) // NW

        def bucket_body(kk, _):
            b = wid + kk * NW
            nbase = b * SPAN

            def zacc(i, _):
                r = i // nvr
                j = i % nvr
                acc[r, pl.ds(j * L, L)] = jnp.zeros((L,), jnp.float32)
                return 0
            lax.fori_loop(0, SPAN * nvr, zacc, 0)

            def zden(r, _):
                den[r, :] = jnp.zeros((L,), jnp.float32)
                return 0
            lax.fori_loop(0, SPAN, zden, 0)

            e0 = bptr_v[pl.ds(b, 1)][0]
            e1 = bptr_v[pl.ds(b + 1, 1)][0]
            ec = e1 - e0
            nch = (ec + C - 1) // C

            def chunk_body(ch, _):
                ebase2 = e0 + ch * C
                _fill_iota(eidx, ebase2, C)
                cpa = pltpu.async_copy(ssrc.at[eidx], ssrc_v, s1)
                cpb = pltpu.async_copy(sdst.at[eidx], sdst_v, s2)
                cpa.wait()
                cpb.wait()
                cp1 = pltpu.async_copy(hfeat.at[ssrc_v], rows, s1)
                cp2 = pltpu.async_copy(asrc.at[ssrc_v], asv, s2)
                cp3 = pltpu.async_copy(adst.at[sdst_v], adv, s3)
                cp1.wait()
                cp2.wait()
                cp3.wait()
                rem = jnp.minimum(C, ec - ch * C)

                def edge_body(e, _):
                    r = sdst_v[pl.ds(e, 1)][0] - nbase
                    ev = asv[e, :] + adv[e, :]
                    ev = jnp.maximum(ev, 0.2 * ev)
                    wv = jnp.exp(ev - cvv)
                    plsc.addupdate(den.at[r], wv)
                    for h in range(H):
                        bc = jnp.full((L,), wv[h])
                        for j in range(F // L):
                            o = h * F + j * L
                            plsc.addupdate(acc.at[r, pl.ds(o, L)],
                                           bc * rows[e, pl.ds(o, L)])
                    return 0
                lax.fori_loop(0, rem, edge_body, 0)
                return 0
            lax.fori_loop(0, nch, chunk_body, 0)

            def flush_body(r, _):
                drow = den[r, :]
                for h in range(H):
                    dv = jnp.full((L,), drow[h]) + 1e-16
                    inv = 1.0 / dv
                    for j in range(F // L):
                        o = h * F + j * L
                        acc[r, pl.ds(o, L)] = acc[r, pl.ds(o, L)] * inv
                return 0
            lax.fori_loop(0, SPAN, flush_body, 0)
            pltpu.sync_copy(acc, out.at[pl.ds(nbase, SPAN)])
            return 0
        lax.fori_loop(0, nown, bucket_body, 0)

    return k


_edge_k1 = _make_edge_kernel(256, 4, 64, 64)
_edge_k2 = _make_edge_kernel(384, 4, 96, 48)
_edge_k3 = _make_edge_kernel(48, 1, 48, 128)


def _prep_body(x_ref, w_ref, asm_ref, adm_ref, h_ref, as_ref, ad_ref, c_ref,
               ms_ref, md_ref):
    i = pl.program_id(0)
    h = jnp.dot(x_ref[...], w_ref[...], preferred_element_type=jnp.float32)
    h_ref[...] = h
    a_s = jnp.dot(h, asm_ref[...], preferred_element_type=jnp.float32)
    a_d = jnp.dot(h, adm_ref[...], preferred_element_type=jnp.float32)
    as_ref[...] = a_s
    ad_ref[...] = a_d
    bs = jnp.max(a_s, axis=0, keepdims=True)
    bd = jnp.max(a_d, axis=0, keepdims=True)

    @pl.when(i == 0)
    def _():
        ms_ref[...] = bs
        md_ref[...] = bd

    @pl.when(i > 0)
    def _():
        ms_ref[...] = jnp.maximum(ms_ref[...], bs)
        md_ref[...] = jnp.maximum(md_ref[...], bd)

    m = ms_ref[...] + md_ref[...]
    c_ref[...] = jnp.maximum(m, 0.2 * m)


def _prep(x, w, asm, adm, K, DOUT):
    return pl.pallas_call(
        _prep_body,
        grid=(GRID,),
        in_specs=[
            pl.BlockSpec((RB, K), lambda i: (i, 0)),
            pl.BlockSpec((K, DOUT), lambda i: (0, 0)),
            pl.BlockSpec((DOUT, 16), lambda i: (0, 0)),
            pl.BlockSpec((DOUT, 16), lambda i: (0, 0)),
        ],
        out_specs=[
            pl.BlockSpec((RB, DOUT), lambda i: (i, 0)),
            pl.BlockSpec((RB, 16), lambda i: (i, 0)),
            pl.BlockSpec((RB, 16), lambda i: (i, 0)),
            pl.BlockSpec((1, 16), lambda i: (0, 0)),
        ],
        out_shape=[
            jax.ShapeDtypeStruct((N, DOUT), jnp.float32),
            jax.ShapeDtypeStruct((N, 16), jnp.float32),
            jax.ShapeDtypeStruct((N, 16), jnp.float32),
            jax.ShapeDtypeStruct((1, 16), jnp.float32),
        ],
        scratch_shapes=[pltpu.VMEM((1, 16), jnp.float32),
                        pltpu.VMEM((1, 16), jnp.float32)],
    )(x, w, asm, adm)


def _stats_body(z_ref, b_ref, sum_ref, sq_ref):
    i = pl.program_id(0)
    t = z_ref[...] + b_ref[...]
    s1 = jnp.sum(t, axis=0, keepdims=True)
    s2 = jnp.sum(t * t, axis=0, keepdims=True)

    @pl.when(i == 0)
    def _():
        sum_ref[...] = s1
        sq_ref[...] = s2

    @pl.when(i > 0)
    def _():
        sum_ref[...] = sum_ref[...] + s1
        sq_ref[...] = sq_ref[...] + s2


def _stats(z, b, D):
    return pl.pallas_call(
        _stats_body,
        grid=(GRID,),
        in_specs=[
            pl.BlockSpec((RB, D), lambda i: (i, 0)),
            pl.BlockSpec((1, D), lambda i: (0, 0)),
        ],
        out_specs=[
            pl.BlockSpec((1, D), lambda i: (0, 0)),
            pl.BlockSpec((1, D), lambda i: (0, 0)),
        ],
        out_shape=[jax.ShapeDtypeStruct((1, D), jnp.float32),
                   jax.ShapeDtypeStruct((1, D), jnp.float32)],
    )(z, b)


def _apply_prep_body(z_ref, b_ref, gw_ref, gb_ref, gms_ref, s1_ref, s2_ref,
                     w_ref, asm_ref, adm_ref, h_ref, as_ref, ad_ref, c_ref,
                     ms_ref, md_ref):
    i = pl.program_id(0)
    t = z_ref[...] + b_ref[...]
    m = s1_ref[...] * (1.0 / N)
    mm = m * gms_ref[...]
    var = s2_ref[...] * (1.0 / N) - 2.0 * mm * m + mm * mm
    y = gw_ref[...] * (t - mm) / jnp.sqrt(var + 1e-5) + gb_ref[...]
    a = jnp.where(y > 0, y, jnp.exp(y) - 1.0)
    h = jnp.dot(a, w_ref[...], preferred_element_type=jnp.float32)
    h_ref[...] = h
    a_s = jnp.dot(h, asm_ref[...], preferred_element_type=jnp.float32)
    a_d = jnp.dot(h, adm_ref[...], preferred_element_type=jnp.float32)
    as_ref[...] = a_s
    ad_ref[...] = a_d
    bs = jnp.max(a_s, axis=0, keepdims=True)
    bd = jnp.max(a_d, axis=0, keepdims=True)

    @pl.when(i == 0)
    def _():
        ms_ref[...] = bs
        md_ref[...] = bd

    @pl.when(i > 0)
    def _():
        ms_ref[...] = jnp.maximum(ms_ref[...], bs)
        md_ref[...] = jnp.maximum(md_ref[...], bd)

    mx = ms_ref[...] + md_ref[...]
    c_ref[...] = jnp.maximum(mx, 0.2 * mx)


def _apply_prep(z, b, gw, gb, gms, s1, s2, w, asm, adm, D, DOUT):
    return pl.pallas_call(
        _apply_prep_body,
        grid=(GRID,),
        in_specs=[
            pl.BlockSpec((RB, D), lambda i: (i, 0)),
            pl.BlockSpec((1, D), lambda i: (0, 0)),
            pl.BlockSpec((1, D), lambda i: (0, 0)),
            pl.BlockSpec((1, D), lambda i: (0, 0)),
            pl.BlockSpec((1, D), lambda i: (0, 0)),
            pl.BlockSpec((1, D), lambda i: (0, 0)),
            pl.BlockSpec((1, D), lambda i: (0, 0)),
            pl.BlockSpec((D, DOUT), lambda i: (0, 0)),
            pl.BlockSpec((DOUT, 16), lambda i: (0, 0)),
            pl.BlockSpec((DOUT, 16), lambda i: (0, 0)),
        ],
        out_specs=[
            pl.BlockSpec((RB, DOUT), lambda i: (i, 0)),
            pl.BlockSpec((RB, 16), lambda i: (i, 0)),
            pl.BlockSpec((RB, 16), lambda i: (i, 0)),
            pl.BlockSpec((1, 16), lambda i: (0, 0)),
        ],
        out_shape=[
            jax.ShapeDtypeStruct((N, DOUT), jnp.float32),
            jax.ShapeDtypeStruct((N, 16), jnp.float32),
            jax.ShapeDtypeStruct((N, 16), jnp.float32),
            jax.ShapeDtypeStruct((1, 16), jnp.float32),
        ],
        scratch_shapes=[pltpu.VMEM((1, 16), jnp.float32),
                        pltpu.VMEM((1, 16), jnp.float32)],
    )(z, b, gw, gb, gms, s1, s2, w, asm, adm)


def _final_body(z_ref, b_ref, o_ref):
    o_ref[...] = z_ref[...] + b_ref[...]


def _final(z, b, D):
    return pl.pallas_call(
        _final_body,
        grid=(GRID,),
        in_specs=[
            pl.BlockSpec((RB, D), lambda i: (i, 0)),
            pl.BlockSpec((1, D), lambda i: (0, 0)),
        ],
        out_specs=pl.BlockSpec((RB, D), lambda i: (i, 0)),
        out_shape=jax.ShapeDtypeStruct((N, D), jnp.float32),
    )(z, b)


def _amap(a, H, F):
    m = jnp.zeros((H * F, 16), jnp.float32)
    for h in range(H):
        m = m.at[h * F:(h + 1) * F, h].set(a[h])
    return m


def kernel(x, edge_index, W1, a1s, a1d, b1, gn1_w, gn1_b, gn1_ms,
           W2, a2s, a2d, b2, gn2_w, gn2_b, gn2_ms, W3, a3s, a3d, b3):
    src = edge_index[0]
    dst = edge_index[1]

    counts = _hist_k(dst)
    off, bptr = _offsets(counts)
    ssrc, sdst = _place_k(src, dst, off)

    xp = jnp.pad(x, ((0, 0), (0, 128 - 11)))
    w1p = jnp.pad(W1, ((0, 128 - 11), (0, 0)))
    h1, as1, ad1, c1 = _prep(xp, w1p, _amap(a1s, 4, 64), _amap(a1d, 4, 64),
                             128, 256)
    z1 = _edge_k1(h1, as1, ad1, ssrc, sdst, bptr, c1)

    s1a, s1b = _stats(z1, b1[None, :], 256)
    h2, as2, ad2, c2 = _apply_prep(z1, b1[None, :], gn1_w[None, :],
                                   gn1_b[None, :], gn1_ms[None, :], s1a, s1b,
                                   W2, _amap(a2s, 4, 96), _amap(a2d, 4, 96),
                                   256, 384)
    z2 = _edge_k2(h2, as2, ad2, ssrc, sdst, bptr, c2)

    s2a, s2b = _stats(z2, b2[None, :], 384)
    h3, as3, ad3, c3 = _apply_prep(z2, b2[None, :], gn2_w[None, :],
                                   gn2_b[None, :], gn2_ms[None, :], s2a, s2b,
                                   W3, _amap(a3s, 1, 48), _amap(a3d, 1, 48),
                                   384, 48)
    z3 = _edge_k3(h3, as3, ad3, ssrc, sdst, bptr, c3)

    return _final(z3, b3[None, :], 48)
